# baseline (speedup 1.0000x reference)
import functools

import jax
import jax.numpy as jnp
from jax import lax
from jax.experimental import pallas as pl
from jax.experimental.pallas import tpu as pltpu
from jax.experimental.pallas import tpu_sc as plsc

NUM_FEATS = 100000
EMB_DIM = 64
BATCH = 4096
N_FIELDS = 26
PHYS_DIM = 128
PHYS_ROWS = NUM_FEATS * EMB_DIM // PHYS_DIM
L = 16
NC, NS = 2, 16
NW = NC * NS
ROWS_PER_W = BATCH // NW
CB = 16
N_CHUNKS = ROWS_PER_W // CB
IDX_PER_W = ROWS_PER_W * N_FIELDS
IDX_PER_CHUNK = CB * N_FIELDS
SUB = 4
IDX_PER_SUB = IDX_PER_CHUNK // SUB
NBUF = 2
DG = 4
N_GROUPS = EMB_DIM // DG


def _fm_body(xf, xh, emb, bias, off, out,
             idx_v, idxh_v, rows_v, bias_v, out_v, off_v, sems):
    wid = lax.axis_index("s") * NC + lax.axis_index("c")
    pltpu.sync_copy(off, off_v)
    pltpu.sync_copy(xf.at[pl.ds(wid * IDX_PER_W, IDX_PER_W)], idx_v)
    pltpu.sync_copy(xh.at[pl.ds(wid * IDX_PER_W, IDX_PER_W)], idxh_v)

    lane = lax.iota(jnp.int32, L)
    lane_b = lane * N_FIELDS
    zero = jnp.zeros((L,), jnp.float32)

    def fire(c, buf):
        for j in range(SUB):
            s0 = c * IDX_PER_CHUNK + j * IDX_PER_SUB
            d0 = buf * IDX_PER_CHUNK + j * IDX_PER_SUB
            pltpu.async_copy(emb.at[idxh_v.at[pl.ds(s0, IDX_PER_SUB)]],
                             rows_v.at[pl.ds(d0, IDX_PER_SUB)], sems.at[buf])
            pltpu.async_copy(bias.at[idx_v.at[pl.ds(s0, IDX_PER_SUB)]],
                             bias_v.at[pl.ds(d0, IDX_PER_SUB)], sems.at[buf])

    def drain(buf):
        for j in range(SUB):
            d0 = buf * IDX_PER_CHUNK + j * IDX_PER_SUB
            pltpu.make_async_copy(
                emb.at[idxh_v.at[pl.ds(0, IDX_PER_SUB)]],
                rows_v.at[pl.ds(d0, IDX_PER_SUB)], sems.at[buf]).wait()
            pltpu.make_async_copy(
                bias.at[idx_v.at[pl.ds(0, IDX_PER_SUB)]],
                bias_v.at[pl.ds(d0, IDX_PER_SUB)], sems.at[buf]).wait()

    def compute(c, buf):
        rbase = buf * IDX_PER_CHUNK + lane_b
        pbase = c * IDX_PER_CHUNK + lane_b

        def per_group(g, carry):
            t, qt = carry
            dvec = lane + g * DG
            didx = [(dvec + dd) & (EMB_DIM - 1) for dd in range(DG)]
            s = [zero] * DG
            q = [zero] * DG
            for f in range(N_FIELDS):
                iv = plsc.load_gather(idx_v, [pbase + f])
                half = (iv & 1) << 6
                ridx = rbase + f
                for dd in range(DG):
                    v = plsc.load_gather(rows_v, [ridx, didx[dd] + half])
                    s[dd] = s[dd] + v
                    q[dd] = q[dd] + v * v
            for dd in range(DG):
                t = t + s[dd] * s[dd]
                qt = qt + q[dd]
            return (t, qt)

        t, qt = lax.fori_loop(0, N_GROUPS, per_group, (zero, zero))
        bsum = zero
        for f in range(N_FIELDS):
            bsum = bsum + plsc.load_gather(bias_v,
                                           [buf * IDX_PER_CHUNK + lane_b + f])
        out_v[...] = 0.5 * (t - qt) + bsum + off_v[...]
        pltpu.sync_copy(out_v, out.at[pl.ds(wid * ROWS_PER_W + c * CB, CB)])

    fire(0, 0)

    def super_body(i, carry):
        c0 = 2 * i
        drain(0)
        fire(c0 + 1, 1)
        compute(c0, 0)
        drain(1)

        @pl.when(i < N_CHUNKS // 2 - 1)
        def _():
            fire(c0 + 2, 0)

        compute(c0 + 1, 1)
        return carry

    lax.fori_loop(0, N_CHUNKS // 2, super_body, 0)


@functools.cache
def _fm_kernel():
    return functools.partial(
        pl.kernel,
        out_type=jax.ShapeDtypeStruct((BATCH,), jnp.float32),
        mesh=plsc.VectorSubcoreMesh(core_axis_name="c", subcore_axis_name="s"),
        compiler_params=pltpu.CompilerParams(
            needs_layout_passes=False, use_tc_tiling_on_sc=True),
        scratch_types=[
            pltpu.VMEM((IDX_PER_W,), jnp.int32),
            pltpu.VMEM((IDX_PER_W,), jnp.int32),
            pltpu.VMEM((NBUF * IDX_PER_CHUNK, PHYS_DIM), jnp.float32),
            pltpu.VMEM((NBUF * IDX_PER_CHUNK,), jnp.float32),
            pltpu.VMEM((L,), jnp.float32),
            pltpu.VMEM((L,), jnp.float32),
            pltpu.SemaphoreType.DMA((NBUF,)),
        ],
    )(_fm_body)


def kernel(X, x_emb_weight, x_bias, offset):
    xf = X.reshape(-1).astype(jnp.int32)
    xh = xf >> 1
    emb2 = x_emb_weight.reshape(PHYS_ROWS, PHYS_DIM)
    off16 = jnp.broadcast_to(offset.astype(jnp.float32), (L,))
    return _fm_kernel()(xf, xh, emb2, x_bias, off16)


# speedup vs baseline: 1.3079x; 1.3079x over previous
import functools

import jax
import jax.numpy as jnp
from jax import lax
from jax.experimental import pallas as pl
from jax.experimental.pallas import tpu as pltpu
from jax.experimental.pallas import tpu_sc as plsc

NUM_FEATS = 100000
EMB_DIM = 64
BATCH = 4096
N_FIELDS = 26
PHYS_DIM = 128
PHYS_ROWS = 50048
TR_BLK = 2176
L = 16
NC, NS = 2, 16
NW = NC * NS
ROWS_PER_W = BATCH // NW
CB = 16
N_CHUNKS = ROWS_PER_W // CB
IDX_PER_W = ROWS_PER_W * N_FIELDS
IDX_PER_CHUNK = CB * N_FIELDS
SUB = 4
IDX_PER_SUB = IDX_PER_CHUNK // SUB
NBUF = 2
DG = 4
N_GROUPS = EMB_DIM // DG


def _fm_body(xf, xh, emb, bias, off, out,
             idx_v, idxh_v, rows_v, bias_v, out_v, off_v, sems):
    wid = lax.axis_index("s") * NC + lax.axis_index("c")
    pltpu.sync_copy(off, off_v)
    pltpu.sync_copy(xf.at[pl.ds(wid * IDX_PER_W, IDX_PER_W)], idx_v)
    pltpu.sync_copy(xh.at[pl.ds(wid * IDX_PER_W, IDX_PER_W)], idxh_v)

    lane = lax.iota(jnp.int32, L)
    lane_b = lane * N_FIELDS
    zero = jnp.zeros((L,), jnp.float32)

    def fire(c, buf):
        for j in range(SUB):
            s0 = c * IDX_PER_CHUNK + j * IDX_PER_SUB
            d0 = buf * IDX_PER_CHUNK + j * IDX_PER_SUB
            pltpu.async_copy(emb.at[idxh_v.at[pl.ds(s0, IDX_PER_SUB)]],
                             rows_v.at[pl.ds(d0, IDX_PER_SUB)], sems.at[buf])
            pltpu.async_copy(bias.at[idx_v.at[pl.ds(s0, IDX_PER_SUB)]],
                             bias_v.at[pl.ds(d0, IDX_PER_SUB)], sems.at[buf])

    def drain(buf):
        for j in range(SUB):
            d0 = buf * IDX_PER_CHUNK + j * IDX_PER_SUB
            pltpu.make_async_copy(
                emb.at[idxh_v.at[pl.ds(0, IDX_PER_SUB)]],
                rows_v.at[pl.ds(d0, IDX_PER_SUB)], sems.at[buf]).wait()
            pltpu.make_async_copy(
                bias.at[idx_v.at[pl.ds(0, IDX_PER_SUB)]],
                bias_v.at[pl.ds(d0, IDX_PER_SUB)], sems.at[buf]).wait()

    def compute(c, buf):
        rbase = buf * IDX_PER_CHUNK + lane_b
        pbase = c * IDX_PER_CHUNK + lane_b

        def per_group(g, carry):
            t, qt = carry
            dvec = lane + g * DG
            didx = [(dvec + dd) & (EMB_DIM - 1) for dd in range(DG)]
            s = [zero] * DG
            q = [zero] * DG
            for f in range(N_FIELDS):
                iv = plsc.load_gather(idx_v, [pbase + f])
                half = jnp.where(iv >= PHYS_ROWS, EMB_DIM, 0)
                ridx = rbase + f
                for dd in range(DG):
                    v = plsc.load_gather(rows_v, [ridx, didx[dd] + half])
                    s[dd] = s[dd] + v
                    q[dd] = q[dd] + v * v
            for dd in range(DG):
                t = t + s[dd] * s[dd]
                qt = qt + q[dd]
            return (t, qt)

        t, qt = lax.fori_loop(0, N_GROUPS, per_group, (zero, zero))
        bsum = zero
        for f in range(N_FIELDS):
            bsum = bsum + plsc.load_gather(bias_v,
                                           [buf * IDX_PER_CHUNK + lane_b + f])
        out_v[...] = 0.5 * (t - qt) + bsum + off_v[...]
        pltpu.sync_copy(out_v, out.at[pl.ds(wid * ROWS_PER_W + c * CB, CB)])

    fire(0, 0)

    def super_body(i, carry):
        c0 = 2 * i
        drain(0)
        fire(c0 + 1, 1)
        compute(c0, 0)
        drain(1)

        @pl.when(i < N_CHUNKS // 2 - 1)
        def _():
            fire(c0 + 2, 0)

        compute(c0 + 1, 1)
        return carry

    lax.fori_loop(0, N_CHUNKS // 2, super_body, 0)


@functools.cache
def _fm_kernel():
    return functools.partial(
        pl.kernel,
        out_type=jax.ShapeDtypeStruct((BATCH,), jnp.float32),
        mesh=plsc.VectorSubcoreMesh(core_axis_name="c", subcore_axis_name="s"),
        compiler_params=pltpu.CompilerParams(
            needs_layout_passes=False, use_tc_tiling_on_sc=True),
        scratch_types=[
            pltpu.VMEM((IDX_PER_W,), jnp.int32),
            pltpu.VMEM((IDX_PER_W,), jnp.int32),
            pltpu.VMEM((NBUF * IDX_PER_CHUNK, PHYS_DIM), jnp.float32),
            pltpu.VMEM((NBUF * IDX_PER_CHUNK,), jnp.float32),
            pltpu.VMEM((L,), jnp.float32),
            pltpu.VMEM((L,), jnp.float32),
            pltpu.SemaphoreType.DMA((NBUF,)),
        ],
    )(_fm_body)


def _tr_body(a_ref, b_ref, o_ref):
    o_ref[:, 0:EMB_DIM] = jnp.transpose(a_ref[...], (1, 0))
    o_ref[:, EMB_DIM:PHYS_DIM] = jnp.transpose(b_ref[...], (1, 0))


@functools.cache
def _tr_kernel():
    nb = PHYS_ROWS // TR_BLK
    return pl.pallas_call(
        _tr_body,
        grid=(nb,),
        in_specs=[
            pl.BlockSpec((EMB_DIM, TR_BLK), lambda i: (0, i)),
            pl.BlockSpec((EMB_DIM, TR_BLK), lambda i: (0, i + nb)),
        ],
        out_specs=pl.BlockSpec((TR_BLK, PHYS_DIM), lambda i: (i, 0)),
        out_shape=jax.ShapeDtypeStruct((PHYS_ROWS, PHYS_DIM), jnp.float32),
    )


def kernel(X, x_emb_weight, x_bias, offset):
    xf = X.reshape(-1).astype(jnp.int32)
    xh = jnp.where(xf >= PHYS_ROWS, xf - PHYS_ROWS, xf)
    emb_t = x_emb_weight.T
    emb2 = _tr_kernel()(emb_t, emb_t)
    off16 = jnp.broadcast_to(offset.astype(jnp.float32), (L,))
    return _fm_kernel()(xf, xh, emb2, x_bias, off16)


# speedup vs baseline: 1.3570x; 1.0376x over previous
import functools

import jax
import jax.numpy as jnp
from jax import lax
from jax.experimental import pallas as pl
from jax.experimental.pallas import tpu as pltpu
from jax.experimental.pallas import tpu_sc as plsc

NUM_FEATS = 100000
EMB_DIM = 64
BATCH = 4096
N_FIELDS = 26
PHYS_DIM = 128
PHYS_ROWS = 50048
TR_BLK = 2944
L = 16
NC, NS = 2, 16
NW = NC * NS
ROWS_PER_W = BATCH // NW
CB = 16
N_CHUNKS = ROWS_PER_W // CB
IDX_PER_W = ROWS_PER_W * N_FIELDS
IDX_PER_CHUNK = CB * N_FIELDS
SUB = 4
IDX_PER_SUB = IDX_PER_CHUNK // SUB
NBUF = 2
DG = 4
N_GROUPS = EMB_DIM // DG


def _fm_body(xf, xh, emb, bias, off, out,
             idx_v, idxh_v, rows_v, bias_v, out_v, off_v, sems):
    wid = lax.axis_index("s") * NC + lax.axis_index("c")
    pltpu.sync_copy(off, off_v)
    pltpu.sync_copy(xf.at[pl.ds(wid * IDX_PER_W, IDX_PER_W)], idx_v)
    pltpu.sync_copy(xh.at[pl.ds(wid * IDX_PER_W, IDX_PER_W)], idxh_v)

    lane = lax.iota(jnp.int32, L)
    lane_b = lane * N_FIELDS
    zero = jnp.zeros((L,), jnp.float32)

    def fire(c, buf):
        for j in range(SUB):
            s0 = c * IDX_PER_CHUNK + j * IDX_PER_SUB
            d0 = buf * IDX_PER_CHUNK + j * IDX_PER_SUB
            pltpu.async_copy(emb.at[idxh_v.at[pl.ds(s0, IDX_PER_SUB)]],
                             rows_v.at[pl.ds(d0, IDX_PER_SUB)], sems.at[buf])
            pltpu.async_copy(bias.at[idx_v.at[pl.ds(s0, IDX_PER_SUB)]],
                             bias_v.at[pl.ds(d0, IDX_PER_SUB)], sems.at[buf])

    def drain(buf):
        for j in range(SUB):
            d0 = buf * IDX_PER_CHUNK + j * IDX_PER_SUB
            pltpu.make_async_copy(
                emb.at[idxh_v.at[pl.ds(0, IDX_PER_SUB)]],
                rows_v.at[pl.ds(d0, IDX_PER_SUB)], sems.at[buf]).wait()
            pltpu.make_async_copy(
                bias.at[idx_v.at[pl.ds(0, IDX_PER_SUB)]],
                bias_v.at[pl.ds(d0, IDX_PER_SUB)], sems.at[buf]).wait()

    def compute(c, buf):
        rbase = buf * IDX_PER_CHUNK + lane_b
        pbase = c * IDX_PER_CHUNK + lane_b

        def per_group(g, carry):
            t, qt = carry
            dvec = lane + g * DG
            didx = [(dvec + dd) & (EMB_DIM - 1) for dd in range(DG)]
            s = [zero] * DG
            q = [zero] * DG
            for f in range(N_FIELDS):
                iv = plsc.load_gather(idx_v, [pbase + f])
                half = jnp.where(iv >= PHYS_ROWS, EMB_DIM, 0)
                ridx = rbase + f
                for dd in range(DG):
                    v = plsc.load_gather(rows_v, [ridx, didx[dd] + half])
                    s[dd] = s[dd] + v
                    q[dd] = q[dd] + v * v
            for dd in range(DG):
                t = t + s[dd] * s[dd]
                qt = qt + q[dd]
            return (t, qt)

        t, qt = lax.fori_loop(0, N_GROUPS, per_group, (zero, zero))
        bsum = zero
        for f in range(N_FIELDS):
            bsum = bsum + plsc.load_gather(bias_v,
                                           [buf * IDX_PER_CHUNK + lane_b + f])
        out_v[...] = 0.5 * (t - qt) + bsum + off_v[...]
        pltpu.sync_copy(out_v, out.at[pl.ds(wid * ROWS_PER_W + c * CB, CB)])

    fire(0, 0)

    def super_body(i, carry):
        c0 = 2 * i
        drain(0)
        fire(c0 + 1, 1)
        compute(c0, 0)
        drain(1)

        @pl.when(i < N_CHUNKS // 2 - 1)
        def _():
            fire(c0 + 2, 0)

        compute(c0 + 1, 1)
        return carry

    lax.fori_loop(0, N_CHUNKS // 2, super_body, 0)


@functools.cache
def _fm_kernel():
    return functools.partial(
        pl.kernel,
        out_type=jax.ShapeDtypeStruct((BATCH,), jnp.float32),
        mesh=plsc.VectorSubcoreMesh(core_axis_name="c", subcore_axis_name="s"),
        compiler_params=pltpu.CompilerParams(
            needs_layout_passes=False, use_tc_tiling_on_sc=True),
        scratch_types=[
            pltpu.VMEM((IDX_PER_W,), jnp.int32),
            pltpu.VMEM((IDX_PER_W,), jnp.int32),
            pltpu.VMEM((NBUF * IDX_PER_CHUNK, PHYS_DIM), jnp.float32),
            pltpu.VMEM((NBUF * IDX_PER_CHUNK,), jnp.float32),
            pltpu.VMEM((L,), jnp.float32),
            pltpu.VMEM((L,), jnp.float32),
            pltpu.SemaphoreType.DMA((NBUF,)),
        ],
    )(_fm_body)


def _tr_body(a_ref, b_ref, o_ref):
    eye = jnp.eye(EMB_DIM, dtype=jnp.float32)
    dn = (((0,), (0,)), ((), ()))
    o_ref[:, 0:EMB_DIM] = lax.dot_general(
        a_ref[...], eye, dn, preferred_element_type=jnp.float32)
    o_ref[:, EMB_DIM:PHYS_DIM] = lax.dot_general(
        b_ref[...], eye, dn, preferred_element_type=jnp.float32)


@functools.cache
def _tr_kernel():
    nb = PHYS_ROWS // TR_BLK
    return pl.pallas_call(
        _tr_body,
        grid=(nb,),
        in_specs=[
            pl.BlockSpec((EMB_DIM, TR_BLK), lambda i: (0, i)),
            pl.BlockSpec((EMB_DIM, TR_BLK), lambda i: (0, i + nb)),
        ],
        out_specs=pl.BlockSpec((TR_BLK, PHYS_DIM), lambda i: (i, 0)),
        out_shape=jax.ShapeDtypeStruct((PHYS_ROWS, PHYS_DIM), jnp.float32),
    )


def kernel(X, x_emb_weight, x_bias, offset):
    xf = X.reshape(-1).astype(jnp.int32)
    xh = jnp.where(xf >= PHYS_ROWS, xf - PHYS_ROWS, xf)
    emb_t = x_emb_weight.T
    emb2 = _tr_kernel()(emb_t, emb_t)
    off16 = jnp.broadcast_to(offset.astype(jnp.float32), (L,))
    return _fm_kernel()(xf, xh, emb2, x_bias, off16)


# speedup vs baseline: 1.3730x; 1.0118x over previous
import functools

import jax
import jax.numpy as jnp
from jax import lax
from jax.experimental import pallas as pl
from jax.experimental.pallas import tpu as pltpu
from jax.experimental.pallas import tpu_sc as plsc

NUM_FEATS = 100000
EMB_DIM = 64
BATCH = 4096
N_FIELDS = 26
PHYS_DIM = 128
PHYS_ROWS = 50048
TR_BLK = 2944
L = 16
NC, NS = 2, 16
NW = NC * NS
ROWS_PER_W = BATCH // NW
CB = 16
N_CHUNKS = ROWS_PER_W // CB
IDX_PER_W = ROWS_PER_W * N_FIELDS
IDX_PER_CHUNK = CB * N_FIELDS
SUB = 4
IDX_PER_SUB = IDX_PER_CHUNK // SUB
NBUF = 2
DG = 8
N_GROUPS = EMB_DIM // DG


def _fm_body(xf, xh, emb, bias, off, out,
             idx_v, idxh_v, rows_v, bias_v, out_v, off_v, sems):
    wid = lax.axis_index("s") * NC + lax.axis_index("c")
    pltpu.sync_copy(off, off_v)
    pltpu.sync_copy(xf.at[pl.ds(wid * IDX_PER_W, IDX_PER_W)], idx_v)
    pltpu.sync_copy(xh.at[pl.ds(wid * IDX_PER_W, IDX_PER_W)], idxh_v)

    lane = lax.iota(jnp.int32, L)
    lane_b = lane * N_FIELDS
    zero = jnp.zeros((L,), jnp.float32)

    def fire(c, buf):
        for j in range(SUB):
            s0 = c * IDX_PER_CHUNK + j * IDX_PER_SUB
            d0 = buf * IDX_PER_CHUNK + j * IDX_PER_SUB
            pltpu.async_copy(emb.at[idxh_v.at[pl.ds(s0, IDX_PER_SUB)]],
                             rows_v.at[pl.ds(d0, IDX_PER_SUB)], sems.at[buf])
            pltpu.async_copy(bias.at[idx_v.at[pl.ds(s0, IDX_PER_SUB)]],
                             bias_v.at[pl.ds(d0, IDX_PER_SUB)], sems.at[buf])

    def drain(buf):
        for j in range(SUB):
            d0 = buf * IDX_PER_CHUNK + j * IDX_PER_SUB
            pltpu.make_async_copy(
                emb.at[idxh_v.at[pl.ds(0, IDX_PER_SUB)]],
                rows_v.at[pl.ds(d0, IDX_PER_SUB)], sems.at[buf]).wait()
            pltpu.make_async_copy(
                bias.at[idx_v.at[pl.ds(0, IDX_PER_SUB)]],
                bias_v.at[pl.ds(d0, IDX_PER_SUB)], sems.at[buf]).wait()

    def compute(c, buf):
        rbase = buf * IDX_PER_CHUNK + lane_b
        pbase = c * IDX_PER_CHUNK + lane_b

        def per_group(g, carry):
            t, qt = carry
            dvec = lane + g * DG
            didx = [(dvec + dd) & (EMB_DIM - 1) for dd in range(DG)]
            s = [zero] * DG
            q = [zero] * DG
            for f in range(N_FIELDS):
                iv = plsc.load_gather(idx_v, [pbase + f])
                half = jnp.where(iv >= PHYS_ROWS, EMB_DIM, 0)
                ridx = rbase + f
                for dd in range(DG):
                    v = plsc.load_gather(rows_v, [ridx, didx[dd] + half])
                    s[dd] = s[dd] + v
                    q[dd] = q[dd] + v * v
            for dd in range(DG):
                t = t + s[dd] * s[dd]
                qt = qt + q[dd]
            return (t, qt)

        t, qt = lax.fori_loop(0, N_GROUPS, per_group, (zero, zero))
        bsum = zero
        for f in range(N_FIELDS):
            bsum = bsum + plsc.load_gather(bias_v,
                                           [buf * IDX_PER_CHUNK + lane_b + f])
        out_v[...] = 0.5 * (t - qt) + bsum + off_v[...]
        pltpu.sync_copy(out_v, out.at[pl.ds(wid * ROWS_PER_W + c * CB, CB)])

    fire(0, 0)

    def super_body(i, carry):
        c0 = 2 * i
        drain(0)
        fire(c0 + 1, 1)

        def h_body(h, hcarry):
            compute(c0 + h, h)

            @pl.when(h == 0)
            def _():
                drain(1)

            @pl.when((h == 0) & (i < N_CHUNKS // 2 - 1))
            def _():
                fire(c0 + 2, 0)

            return hcarry

        lax.fori_loop(0, 2, h_body, 0)
        return carry

    lax.fori_loop(0, N_CHUNKS // 2, super_body, 0)


@functools.cache
def _fm_kernel():
    return functools.partial(
        pl.kernel,
        out_type=jax.ShapeDtypeStruct((BATCH,), jnp.float32),
        mesh=plsc.VectorSubcoreMesh(core_axis_name="c", subcore_axis_name="s"),
        compiler_params=pltpu.CompilerParams(
            needs_layout_passes=False, use_tc_tiling_on_sc=True),
        scratch_types=[
            pltpu.VMEM((IDX_PER_W,), jnp.int32),
            pltpu.VMEM((IDX_PER_W,), jnp.int32),
            pltpu.VMEM((NBUF * IDX_PER_CHUNK, PHYS_DIM), jnp.float32),
            pltpu.VMEM((NBUF * IDX_PER_CHUNK,), jnp.float32),
            pltpu.VMEM((L,), jnp.float32),
            pltpu.VMEM((L,), jnp.float32),
            pltpu.SemaphoreType.DMA((NBUF,)),
        ],
    )(_fm_body)


def _tr_body(a_ref, b_ref, o_ref):
    eye = jnp.eye(EMB_DIM, dtype=jnp.float32)
    dn = (((0,), (0,)), ((), ()))
    o_ref[:, 0:EMB_DIM] = lax.dot_general(
        a_ref[...], eye, dn, preferred_element_type=jnp.float32)
    o_ref[:, EMB_DIM:PHYS_DIM] = lax.dot_general(
        b_ref[...], eye, dn, preferred_element_type=jnp.float32)


@functools.cache
def _tr_kernel():
    nb = PHYS_ROWS // TR_BLK
    return pl.pallas_call(
        _tr_body,
        grid=(nb,),
        in_specs=[
            pl.BlockSpec((EMB_DIM, TR_BLK), lambda i: (0, i)),
            pl.BlockSpec((EMB_DIM, TR_BLK), lambda i: (0, i + nb)),
        ],
        out_specs=pl.BlockSpec((TR_BLK, PHYS_DIM), lambda i: (i, 0)),
        out_shape=jax.ShapeDtypeStruct((PHYS_ROWS, PHYS_DIM), jnp.float32),
    )


def kernel(X, x_emb_weight, x_bias, offset):
    xf = X.reshape(-1).astype(jnp.int32)
    xh = jnp.where(xf >= PHYS_ROWS, xf - PHYS_ROWS, xf)
    emb_t = x_emb_weight.T
    emb2 = _tr_kernel()(emb_t, emb_t)
    off16 = jnp.broadcast_to(offset.astype(jnp.float32), (L,))
    return _fm_kernel()(xf, xh, emb2, x_bias, off16)
